# baseline (speedup 1.0000x reference)
import functools

import jax
import jax.numpy as jnp
from jax import lax
from jax.experimental import pallas as pl
from jax.experimental.pallas import tpu as pltpu
from jax.experimental.pallas import tpu_sc as plsc

N = 10000
E = 160000
D = 256

NC = 2
NS = 16
NW = NC * NS

NSLAB = 5
ES = E // NSLAB
EPW = ES // NW
GCH = 40
GNCH = EPW // GCH

COLS = D // NC
EPT = ES // NS
SCH = 80
SNCH = EPT // SCH
NP = 10240
RPT = NP // NS
RPT_LAST = N - (NS - 1) * RPT

_mesh = plsc.VectorSubcoreMesh(
    core_axis_name="c", subcore_axis_name="s", num_cores=NC, num_subcores=NS)


DP = D // 2


@functools.partial(
    pl.kernel,
    out_type=[jax.ShapeDtypeStruct((ES, DP), jnp.float32),
              jax.ShapeDtypeStruct((ES, DP), jnp.float32)],
    mesh=_mesh,
    scratch_types=[
        pltpu.VMEM((2, GCH), jnp.int32),
        pltpu.VMEM((2, GCH), jnp.int32),
        pltpu.VMEM((2, GCH, DP), jnp.float32),
        pltpu.VMEM((2, GCH, DP), jnp.float32),
        pltpu.SemaphoreType.DMA((2,)),
        pltpu.SemaphoreType.DMA((2,)),
        pltpu.SemaphoreType.DMA((2,)),
        pltpu.SemaphoreType.DMA((2,)),
    ],
)
def _sc_gather(nodes_hbm, senders_hbm, receivers_hbm, sf_hbm, rf_hbm,
               sidx_v, ridx_v, srows_v, rrows_v, gs, gr, ws, wr):
    wid = lax.axis_index("s") * NC + lax.axis_index("c")
    base = wid * EPW

    def load_idx(j, slot):
        off = base + j * GCH
        pltpu.sync_copy(senders_hbm.at[pl.ds(off, GCH)], sidx_v.at[slot])
        pltpu.sync_copy(receivers_hbm.at[pl.ds(off, GCH)], ridx_v.at[slot])

    def start_gather(slot):
        pltpu.async_copy(nodes_hbm.at[sidx_v.at[slot]], srows_v.at[slot],
                         gs.at[slot])
        pltpu.async_copy(nodes_hbm.at[ridx_v.at[slot]], rrows_v.at[slot],
                         gr.at[slot])

    def wait_gather(slot):
        pltpu.make_async_copy(nodes_hbm.at[sidx_v.at[slot]], srows_v.at[slot],
                              gs.at[slot]).wait()
        pltpu.make_async_copy(nodes_hbm.at[ridx_v.at[slot]], rrows_v.at[slot],
                              gr.at[slot]).wait()

    def start_write(j, slot):
        off = base + j * GCH
        pltpu.async_copy(srows_v.at[slot], sf_hbm.at[pl.ds(off, GCH)],
                         ws.at[slot])
        pltpu.async_copy(rrows_v.at[slot], rf_hbm.at[pl.ds(off, GCH)],
                         wr.at[slot])

    def wait_write(j, slot):
        off = base + j * GCH
        pltpu.make_async_copy(srows_v.at[slot], sf_hbm.at[pl.ds(off, GCH)],
                              ws.at[slot]).wait()
        pltpu.make_async_copy(rrows_v.at[slot], rf_hbm.at[pl.ds(off, GCH)],
                              wr.at[slot]).wait()

    load_idx(0, 0)
    start_gather(0)
    load_idx(1, 1)
    start_gather(1)
    wait_gather(0)
    start_write(0, 0)

    def body(j, carry):
        slot = lax.rem(j, 2)
        other = 1 - slot
        wait_write(j - 2, slot)
        load_idx(j, slot)
        start_gather(slot)
        wait_gather(other)
        start_write(j - 1, other)
        return carry

    lax.fori_loop(2, GNCH, body, 0)
    last = (GNCH - 1) % 2
    wait_gather(last)
    start_write(GNCH - 1, last)
    wait_write(GNCH - 2, 1 - last)
    wait_write(GNCH - 1, last)


@functools.partial(
    pl.kernel,
    out_type=jax.ShapeDtypeStruct((N, D), jnp.float32),
    mesh=_mesh,
    scratch_types=[
        pltpu.VMEM((2, SCH), jnp.int32),
        pltpu.VMEM((2, SCH, COLS), jnp.float32),
        pltpu.VMEM_SHARED((NP, COLS), jnp.float32),
        pltpu.SemaphoreType.DMA((2,)),
        pltpu.SemaphoreType.DMA((2,)),
    ],
)
def _sc_scatter(ne0, ne1, ne2, ne3, ne4, receivers_hbm, zeros_hbm, agg_hbm,
                idx_v, rows_v, acc_sh, ls, as_):
    sid = lax.axis_index("s")
    cid = lax.axis_index("c")
    col0 = cid * COLS
    row0 = sid * RPT
    pltpu.sync_copy(zeros_hbm, acc_sh.at[pl.ds(row0, RPT)])
    plsc.subcore_barrier()

    for s, ne_hbm in enumerate((ne0, ne1, ne2, ne3, ne4)):
        ebase = sid * EPT
        ibase = s * ES + sid * EPT

        def load(j, slot):
            pltpu.sync_copy(receivers_hbm.at[pl.ds(ibase + j * SCH, SCH)],
                            idx_v.at[slot])
            pltpu.async_copy(
                ne_hbm.at[pl.ds(ebase + j * SCH, SCH), pl.ds(col0, COLS)],
                rows_v.at[slot], ls.at[slot])

        def wait_load(j, slot):
            pltpu.make_async_copy(
                ne_hbm.at[pl.ds(ebase + j * SCH, SCH), pl.ds(col0, COLS)],
                rows_v.at[slot], ls.at[slot]).wait()

        def start_add(slot):
            pltpu.async_copy(rows_v.at[slot], acc_sh.at[idx_v.at[slot]],
                             as_.at[slot], add=True)

        def wait_add(slot):
            pltpu.make_async_copy(rows_v.at[slot], acc_sh.at[idx_v.at[slot]],
                                  as_.at[slot]).wait()

        load(0, 0)
        load(1, 1)
        wait_load(0, 0)
        start_add(0)

        def body(j, carry):
            slot = lax.rem(j, 2)
            other = 1 - slot
            wait_add(slot)
            load(j, slot)
            wait_load(j - 1, other)
            start_add(other)
            return carry

        lax.fori_loop(2, SNCH, body, 0)
        last = (SNCH - 1) % 2
        wait_add(1 - last)
        wait_load(SNCH - 1, last)
        start_add(last)
        wait_add(last)
    plsc.subcore_barrier()

    @pl.when(sid < NS - 1)
    def _store_full():
        pltpu.sync_copy(acc_sh.at[pl.ds(row0, RPT)],
                        agg_hbm.at[pl.ds(row0, RPT), pl.ds(col0, COLS)])

    @pl.when(sid == NS - 1)
    def _store_last():
        pltpu.sync_copy(acc_sh.at[pl.ds((NS - 1) * RPT, RPT_LAST)],
                        agg_hbm.at[pl.ds((NS - 1) * RPT, RPT_LAST),
                                   pl.ds(col0, COLS)])


BE = 1600
BN = 1000


def _unpack_bf16(packed_f32):
    pi = lax.bitcast_convert_type(packed_f32, jnp.int32)
    lo = lax.bitcast_convert_type(pi << 16, jnp.float32)
    hi = lax.bitcast_convert_type(pi & jnp.int32(-65536), jnp.float32)
    return lo.astype(jnp.bfloat16), hi.astype(jnp.bfloat16)


def _edge_mlp_body(sfp, rfp, ef, W1cat, b1, W2, b2, W3, b3, g, beta, ne, oe):
    bf16 = jnp.bfloat16
    f32 = jnp.float32
    se, so = _unpack_bf16(sfp[...])
    re_, ro = _unpack_bf16(rfp[...])
    xin = jnp.concatenate([se, so, re_, ro, ef[...].astype(bf16)], axis=-1)
    x = jnp.dot(xin, W1cat[...], preferred_element_type=f32)
    h = jnp.maximum(x + b1[...], 0.0).astype(bf16)
    h = jnp.maximum(
        jnp.dot(h, W2[...], preferred_element_type=f32) + b2[...],
        0.0).astype(bf16)
    h = jnp.dot(h, W3[...], preferred_element_type=f32) + b3[...]
    mu = jnp.mean(h, axis=-1, keepdims=True)
    c = h - mu
    var = jnp.mean(c * c, axis=-1, keepdims=True)
    y = c * lax.rsqrt(var + 1e-5) * g[...] + beta[...]
    ne[...] = y
    oe[...] = y + ef[...]


def _node_mlp_body(nf, agg, W1, b1, W2, b2, W3, b3, g, beta, on):
    x = jnp.dot(nf[...], W1[0:D, :], preferred_element_type=jnp.float32)
    x = x + jnp.dot(agg[...], W1[D:2 * D, :], preferred_element_type=jnp.float32)
    h = jnp.maximum(x + b1[...], 0.0)
    h = jnp.maximum(
        jnp.dot(h, W2[...], preferred_element_type=jnp.float32) + b2[...], 0.0)
    h = jnp.dot(h, W3[...], preferred_element_type=jnp.float32) + b3[...]
    mu = jnp.mean(h, axis=-1, keepdims=True)
    c = h - mu
    var = jnp.mean(c * c, axis=-1, keepdims=True)
    y = c * lax.rsqrt(var + 1e-5) * g[...] + beta[...]
    on[...] = y + nf[...]


def _row_spec(b):
    return pl.BlockSpec((b, D), lambda i: (i, 0))


def _full_spec(r):
    return pl.BlockSpec((r, D), lambda i: (0, 0))


_edge_call = pl.pallas_call(
    _edge_mlp_body,
    grid=(ES // BE,),
    in_specs=[
        pl.BlockSpec((BE, DP), lambda i: (i, 0)),
        pl.BlockSpec((BE, DP), lambda i: (i, 0)),
        _row_spec(BE),
        _full_spec(3 * D), _full_spec(1),
        _full_spec(D), _full_spec(1),
        _full_spec(D), _full_spec(1),
        _full_spec(1), _full_spec(1),
    ],
    out_specs=[_row_spec(BE)] * 2,
    out_shape=[jax.ShapeDtypeStruct((ES, D), jnp.float32),
               jax.ShapeDtypeStruct((ES, D), jnp.float32)],
)

_node_call = pl.pallas_call(
    _node_mlp_body,
    grid=(N // BN,),
    in_specs=[_row_spec(BN)] * 2 + [
        _full_spec(2 * D), _full_spec(1),
        _full_spec(D), _full_spec(1),
        _full_spec(D), _full_spec(1),
        _full_spec(1), _full_spec(1),
    ],
    out_specs=_row_spec(BN),
    out_shape=jax.ShapeDtypeStruct((N, D), jnp.float32),
)


def kernel(node_features, edge_features, senders, receivers,
           eW1, eb1, eW2, eb2, eW3, eb3, eg, ebeta,
           nW1, nb1, nW2, nb2, nW3, nb3, ng, nbeta):
    r1 = lambda v: v.reshape(1, D)
    bf16 = jnp.bfloat16
    ni = lax.bitcast_convert_type(node_features, jnp.int32)
    rnd = lambda x: x + jnp.int32(0x7FFF) + ((x >> 16) & jnp.int32(1))
    lo = (rnd(ni[:, :DP]) >> 16) & jnp.int32(0xFFFF)
    hi = rnd(ni[:, DP:]) & jnp.int32(-65536)
    node_p = lax.bitcast_convert_type(lo | hi, jnp.float32)
    eW1_16, eW2_16, eW3_16 = eW1.astype(bf16), eW2.astype(bf16), eW3.astype(bf16)
    ne_slabs, oe_slabs = [], []
    for k in range(NSLAB):
        sfp, rfp = _sc_gather(node_p,
                              lax.slice_in_dim(senders, k * ES, (k + 1) * ES),
                              lax.slice_in_dim(receivers, k * ES, (k + 1) * ES))
        ne_k, oe_k = _edge_call(
            sfp, rfp, lax.slice_in_dim(edge_features, k * ES, (k + 1) * ES),
            eW1_16, r1(eb1), eW2_16, r1(eb2),
            eW3_16, r1(eb3), r1(eg), r1(ebeta))
        ne_slabs.append(ne_k)
        oe_slabs.append(oe_k)
    out_edges = jnp.concatenate(oe_slabs, axis=0)
    zeros = jnp.zeros((RPT, COLS), jnp.float32)
    agg = _sc_scatter(*ne_slabs, receivers, zeros)
    out_nodes = _node_call(
        node_features, agg,
        nW1, r1(nb1), nW2, r1(nb2), nW3, r1(nb3), r1(ng), r1(nbeta))
    return (out_nodes, out_edges)


# speedup vs baseline: 1.1223x; 1.1223x over previous
import functools

import jax
import jax.numpy as jnp
from jax import lax
from jax.experimental import pallas as pl
from jax.experimental.pallas import tpu as pltpu
from jax.experimental.pallas import tpu_sc as plsc

N = 10000
E = 160000
D = 256

NC = 2
NS = 16
NW = NC * NS

EPW = E // NW
GCH = 40
GNCH = EPW // GCH

COLS = D // NC
EPT = E // NS
SCH = 80
SNCH = EPT // SCH
NP = 10240
RPT = NP // NS
RPT_LAST = N - (NS - 1) * RPT

_mesh = plsc.VectorSubcoreMesh(
    core_axis_name="c", subcore_axis_name="s", num_cores=NC, num_subcores=NS)


DP = D // 2


@functools.partial(
    pl.kernel,
    out_type=[jax.ShapeDtypeStruct((E, DP), jnp.float32),
              jax.ShapeDtypeStruct((E, DP), jnp.float32)],
    mesh=_mesh,
    scratch_types=[
        pltpu.VMEM((2, GCH), jnp.int32),
        pltpu.VMEM((2, GCH), jnp.int32),
        pltpu.VMEM((2, GCH, DP), jnp.float32),
        pltpu.VMEM((2, GCH, DP), jnp.float32),
        pltpu.SemaphoreType.DMA((2,)),
        pltpu.SemaphoreType.DMA((2,)),
        pltpu.SemaphoreType.DMA((2,)),
        pltpu.SemaphoreType.DMA((2,)),
    ],
)
def _sc_gather(nodes_hbm, senders_hbm, receivers_hbm, sf_hbm, rf_hbm,
               sidx_v, ridx_v, srows_v, rrows_v, gs, gr, ws, wr):
    wid = lax.axis_index("s") * NC + lax.axis_index("c")
    base = wid * EPW

    def load_idx(j, slot):
        off = base + j * GCH
        pltpu.sync_copy(senders_hbm.at[pl.ds(off, GCH)], sidx_v.at[slot])
        pltpu.sync_copy(receivers_hbm.at[pl.ds(off, GCH)], ridx_v.at[slot])

    def start_gather(slot):
        pltpu.async_copy(nodes_hbm.at[sidx_v.at[slot]], srows_v.at[slot],
                         gs.at[slot])
        pltpu.async_copy(nodes_hbm.at[ridx_v.at[slot]], rrows_v.at[slot],
                         gr.at[slot])

    def wait_gather(slot):
        pltpu.make_async_copy(nodes_hbm.at[sidx_v.at[slot]], srows_v.at[slot],
                              gs.at[slot]).wait()
        pltpu.make_async_copy(nodes_hbm.at[ridx_v.at[slot]], rrows_v.at[slot],
                              gr.at[slot]).wait()

    def start_write(j, slot):
        off = base + j * GCH
        pltpu.async_copy(srows_v.at[slot], sf_hbm.at[pl.ds(off, GCH)],
                         ws.at[slot])
        pltpu.async_copy(rrows_v.at[slot], rf_hbm.at[pl.ds(off, GCH)],
                         wr.at[slot])

    def wait_write(j, slot):
        off = base + j * GCH
        pltpu.make_async_copy(srows_v.at[slot], sf_hbm.at[pl.ds(off, GCH)],
                              ws.at[slot]).wait()
        pltpu.make_async_copy(rrows_v.at[slot], rf_hbm.at[pl.ds(off, GCH)],
                              wr.at[slot]).wait()

    load_idx(0, 0)
    start_gather(0)
    load_idx(1, 1)
    start_gather(1)
    wait_gather(0)
    start_write(0, 0)

    def body(j, carry):
        slot = lax.rem(j, 2)
        other = 1 - slot
        wait_write(j - 2, slot)
        load_idx(j, slot)
        start_gather(slot)
        wait_gather(other)
        start_write(j - 1, other)
        return carry

    lax.fori_loop(2, GNCH, body, 0)
    last = (GNCH - 1) % 2
    wait_gather(last)
    start_write(GNCH - 1, last)
    wait_write(GNCH - 2, 1 - last)
    wait_write(GNCH - 1, last)


@functools.partial(
    pl.kernel,
    out_type=jax.ShapeDtypeStruct((N, D), jnp.float32),
    mesh=_mesh,
    scratch_types=[
        pltpu.VMEM((2, SCH), jnp.int32),
        pltpu.VMEM((2, SCH, COLS), jnp.float32),
        pltpu.VMEM_SHARED((NP, COLS), jnp.float32),
        pltpu.SemaphoreType.DMA((2,)),
        pltpu.SemaphoreType.DMA((2,)),
    ],
)
def _sc_scatter(new_edge_hbm, receivers_hbm, zeros_hbm, agg_hbm,
                idx_v, rows_v, acc_sh, ls, as_):
    sid = lax.axis_index("s")
    cid = lax.axis_index("c")
    col0 = cid * COLS
    row0 = sid * RPT
    ebase = sid * EPT
    pltpu.sync_copy(zeros_hbm, acc_sh.at[pl.ds(row0, RPT)])
    plsc.subcore_barrier()

    def load(j, slot):
        off = ebase + j * SCH
        pltpu.sync_copy(receivers_hbm.at[pl.ds(off, SCH)], idx_v.at[slot])
        pltpu.async_copy(new_edge_hbm.at[pl.ds(off, SCH), pl.ds(col0, COLS)],
                         rows_v.at[slot], ls.at[slot])

    def wait_load(j, slot):
        off = ebase + j * SCH
        pltpu.make_async_copy(
            new_edge_hbm.at[pl.ds(off, SCH), pl.ds(col0, COLS)],
            rows_v.at[slot], ls.at[slot]).wait()

    def start_add(slot):
        pltpu.async_copy(rows_v.at[slot], acc_sh.at[idx_v.at[slot]],
                         as_.at[slot], add=True)

    def wait_add(slot):
        pltpu.make_async_copy(rows_v.at[slot], acc_sh.at[idx_v.at[slot]],
                              as_.at[slot]).wait()

    load(0, 0)
    load(1, 1)
    wait_load(0, 0)
    start_add(0)

    def body(j, carry):
        slot = lax.rem(j, 2)
        other = 1 - slot
        wait_add(slot)
        load(j, slot)
        wait_load(j - 1, other)
        start_add(other)
        return carry

    lax.fori_loop(2, SNCH, body, 0)
    last = (SNCH - 1) % 2
    wait_add(1 - last)
    wait_load(SNCH - 1, last)
    start_add(last)
    wait_add(last)
    plsc.subcore_barrier()

    @pl.when(sid < NS - 1)
    def _store_full():
        pltpu.sync_copy(acc_sh.at[pl.ds(row0, RPT)],
                        agg_hbm.at[pl.ds(row0, RPT), pl.ds(col0, COLS)])

    @pl.when(sid == NS - 1)
    def _store_last():
        pltpu.sync_copy(acc_sh.at[pl.ds((NS - 1) * RPT, RPT_LAST)],
                        agg_hbm.at[pl.ds((NS - 1) * RPT, RPT_LAST),
                                   pl.ds(col0, COLS)])


BE = 1600
BN = 1000


def _unpack_bf16(packed_f32):
    pi = lax.bitcast_convert_type(packed_f32, jnp.int32)
    lo = lax.bitcast_convert_type(pi << 16, jnp.float32)
    hi = lax.bitcast_convert_type(pi & jnp.int32(-65536), jnp.float32)
    return lo.astype(jnp.bfloat16), hi.astype(jnp.bfloat16)


def _edge_mlp_body(sfp, rfp, ef, W1cat, b1, W2, b2, W3, b3, g, beta, ne, oe):
    bf16 = jnp.bfloat16
    f32 = jnp.float32
    se, so = _unpack_bf16(sfp[...])
    re_, ro = _unpack_bf16(rfp[...])
    xin = jnp.concatenate([se, so, re_, ro, ef[...].astype(bf16)], axis=-1)
    x = jnp.dot(xin, W1cat[...], preferred_element_type=f32)
    h = jnp.maximum(x + b1[...], 0.0).astype(bf16)
    h = jnp.maximum(
        jnp.dot(h, W2[...], preferred_element_type=f32) + b2[...],
        0.0).astype(bf16)
    h = jnp.dot(h, W3[...], preferred_element_type=f32) + b3[...]
    mu = jnp.mean(h, axis=-1, keepdims=True)
    c = h - mu
    var = jnp.mean(c * c, axis=-1, keepdims=True)
    y = c * lax.rsqrt(var + 1e-5) * g[...] + beta[...]
    ne[...] = y
    oe[...] = y + ef[...]


def _node_mlp_body(nf, agg, W1, b1, W2, b2, W3, b3, g, beta, on):
    x = jnp.dot(nf[...], W1[0:D, :], preferred_element_type=jnp.float32)
    x = x + jnp.dot(agg[...], W1[D:2 * D, :], preferred_element_type=jnp.float32)
    h = jnp.maximum(x + b1[...], 0.0)
    h = jnp.maximum(
        jnp.dot(h, W2[...], preferred_element_type=jnp.float32) + b2[...], 0.0)
    h = jnp.dot(h, W3[...], preferred_element_type=jnp.float32) + b3[...]
    mu = jnp.mean(h, axis=-1, keepdims=True)
    c = h - mu
    var = jnp.mean(c * c, axis=-1, keepdims=True)
    y = c * lax.rsqrt(var + 1e-5) * g[...] + beta[...]
    on[...] = y + nf[...]


def _row_spec(b):
    return pl.BlockSpec((b, D), lambda i: (i, 0))


def _full_spec(r):
    return pl.BlockSpec((r, D), lambda i: (0, 0))


_edge_call = pl.pallas_call(
    _edge_mlp_body,
    grid=(E // BE,),
    in_specs=[
        pl.BlockSpec((BE, DP), lambda i: (i, 0)),
        pl.BlockSpec((BE, DP), lambda i: (i, 0)),
        _row_spec(BE),
        _full_spec(3 * D), _full_spec(1),
        _full_spec(D), _full_spec(1),
        _full_spec(D), _full_spec(1),
        _full_spec(1), _full_spec(1),
    ],
    out_specs=[_row_spec(BE)] * 2,
    out_shape=[jax.ShapeDtypeStruct((E, D), jnp.float32),
               jax.ShapeDtypeStruct((E, D), jnp.float32)],
)

_node_call = pl.pallas_call(
    _node_mlp_body,
    grid=(N // BN,),
    in_specs=[_row_spec(BN)] * 2 + [
        _full_spec(2 * D), _full_spec(1),
        _full_spec(D), _full_spec(1),
        _full_spec(D), _full_spec(1),
        _full_spec(1), _full_spec(1),
    ],
    out_specs=_row_spec(BN),
    out_shape=jax.ShapeDtypeStruct((N, D), jnp.float32),
)


def kernel(node_features, edge_features, senders, receivers,
           eW1, eb1, eW2, eb2, eW3, eb3, eg, ebeta,
           nW1, nb1, nW2, nb2, nW3, nb3, ng, nbeta):
    r1 = lambda v: v.reshape(1, D)
    bf16 = jnp.bfloat16
    ni = lax.bitcast_convert_type(node_features, jnp.int32)
    rnd = lambda x: x + jnp.int32(0x7FFF) + ((x >> 16) & jnp.int32(1))
    lo = (rnd(ni[:, :DP]) >> 16) & jnp.int32(0xFFFF)
    hi = rnd(ni[:, DP:]) & jnp.int32(-65536)
    node_p = lax.bitcast_convert_type(lo | hi, jnp.float32)
    sfp, rfp = _sc_gather(node_p, senders, receivers)
    new_edge, out_edges = _edge_call(
        sfp, rfp, edge_features,
        eW1.astype(bf16), r1(eb1), eW2.astype(bf16), r1(eb2),
        eW3.astype(bf16), r1(eb3), r1(eg), r1(ebeta))
    zeros = jnp.zeros((RPT, COLS), jnp.float32)
    agg = _sc_scatter(new_edge, receivers, zeros)
    out_nodes = _node_call(
        node_features, agg,
        nW1, r1(nb1), nW2, r1(nb2), nW3, r1(nb3), r1(ng), r1(nbeta))
    return (out_nodes, out_edges)
